# Initial kernel scaffold; baseline (speedup 1.0000x reference)
import jax
import jax.numpy as jnp
from jax.experimental import pallas as pl


def kernel(x, W1, b1, W2, b2, lambda_pre):
    raise NotImplementedError("write your pallas kernel here")


# speedup vs baseline: 15.6948x; 15.6948x over previous
import jax
import jax.numpy as jnp
import numpy as np
from jax.experimental import pallas as pl
from jax.experimental.pallas import tpu as pltpu

_DIMIN = 1024
_NUMNEURO = 4096
_DIMOUT = 1024
_TOPK = 64
_BM = 256

_INT_MIN = np.int32(-(2**31))
_MAG_MASK = np.int32(0x7FFFFFFF)


def _body(lam_ref, x_ref, w1_ref, b1_ref, w2_ref, b2_ref, o_ref):
    lam = lam_ref[0, 0]
    xint = (
        jnp.dot(x_ref[...], w1_ref[...], preferred_element_type=jnp.float32)
        + b1_ref[...]
    )
    keys = jax.lax.bitcast_convert_type(xint, jnp.int32)
    keys = jnp.where(keys < 0, keys ^ _MAG_MASK, keys)
    cnt = jnp.sum((keys >= 0).astype(jnp.int32), axis=1, keepdims=True)
    prefix = jnp.where(cnt >= _TOPK, np.int32(0), _INT_MIN)
    for b in range(30, -1, -1):
        cand = prefix + np.int32(1 << b)
        cnt = jnp.sum((keys >= cand).astype(jnp.int32), axis=1, keepdims=True)
        prefix = jnp.where(cnt >= _TOPK, cand, prefix)
    masked = jnp.where(keys >= prefix, xint * lam, 0.0)
    o_ref[...] = (
        jnp.dot(masked, w2_ref[...], preferred_element_type=jnp.float32)
        + b2_ref[...]
    )


def kernel(x, W1, b1, W2, b2, lambda_pre):
    n = x.shape[0]
    lam = jax.nn.softplus(lambda_pre).reshape(1, 1)
    grid = (n // _BM,)
    return pl.pallas_call(
        _body,
        grid=grid,
        in_specs=[
            pl.BlockSpec(memory_space=pltpu.SMEM),
            pl.BlockSpec((_BM, _DIMIN), lambda i: (i, 0)),
            pl.BlockSpec((_DIMIN, _NUMNEURO), lambda i: (0, 0)),
            pl.BlockSpec((1, _NUMNEURO), lambda i: (0, 0)),
            pl.BlockSpec((_NUMNEURO, _DIMOUT), lambda i: (0, 0)),
            pl.BlockSpec((1, _DIMOUT), lambda i: (0, 0)),
        ],
        out_specs=pl.BlockSpec((_BM, _DIMOUT), lambda i: (i, 0)),
        out_shape=jax.ShapeDtypeStruct((n, _DIMOUT), jnp.float32),
    )(lam, x, W1.T, b1.reshape(1, -1), W2.T, b2.reshape(1, -1))


# speedup vs baseline: 15.7223x; 1.0018x over previous
import jax
import jax.numpy as jnp
import numpy as np
from jax.experimental import pallas as pl
from jax.experimental.pallas import tpu as pltpu

_DIMIN = 1024
_NUMNEURO = 4096
_DIMOUT = 1024
_TOPK = 64
_BM = 256

_INT_MIN = np.int32(-(2**31))
_MAG_MASK = np.int32(0x7FFFFFFF)


def _body(lam_ref, x_ref, w1_ref, b1_ref, w2_ref, b2_ref, o_ref):
    lam = lam_ref[0, 0]
    xint = (
        jnp.dot(x_ref[...], w1_ref[...], preferred_element_type=jnp.float32)
        + b1_ref[...]
    )
    keys = jax.lax.bitcast_convert_type(xint, jnp.int32)
    keys = jnp.where(keys < 0, keys ^ _MAG_MASK, keys)
    cnt = jnp.sum((keys >= 0).astype(jnp.int32), axis=1, keepdims=True)
    prefix = jnp.where(cnt >= _TOPK, np.int32(0), _INT_MIN)
    for b in range(30, -1, -1):
        cand = prefix + np.int32(1 << b)
        cnt = jnp.sum((keys >= cand).astype(jnp.int32), axis=1, keepdims=True)
        prefix = jnp.where(cnt >= _TOPK, cand, prefix)
    masked = jnp.where(keys >= prefix, xint * lam, 0.0).astype(jnp.bfloat16)
    o_ref[...] = (
        jnp.dot(masked, w2_ref[...], preferred_element_type=jnp.float32)
        + b2_ref[...]
    )


def kernel(x, W1, b1, W2, b2, lambda_pre):
    n = x.shape[0]
    lam = jax.nn.softplus(lambda_pre).reshape(1, 1)
    grid = (n // _BM,)
    return pl.pallas_call(
        _body,
        grid=grid,
        in_specs=[
            pl.BlockSpec(memory_space=pltpu.SMEM),
            pl.BlockSpec((_BM, _DIMIN), lambda i: (i, 0)),
            pl.BlockSpec((_DIMIN, _NUMNEURO), lambda i: (0, 0)),
            pl.BlockSpec((1, _NUMNEURO), lambda i: (0, 0)),
            pl.BlockSpec((_NUMNEURO, _DIMOUT), lambda i: (0, 0)),
            pl.BlockSpec((1, _DIMOUT), lambda i: (0, 0)),
        ],
        out_specs=pl.BlockSpec((_BM, _DIMOUT), lambda i: (i, 0)),
        out_shape=jax.ShapeDtypeStruct((n, _DIMOUT), jnp.float32),
    )(
        lam,
        x.astype(jnp.bfloat16),
        W1.T.astype(jnp.bfloat16),
        b1.reshape(1, -1),
        W2.T.astype(jnp.bfloat16),
        b2.reshape(1, -1),
    )
